# baseline (speedup 1.0000x reference)
import functools

import jax
import jax.numpy as jnp
from jax import lax
from jax.experimental import pallas as pl
from jax.experimental.pallas import tpu as pltpu
from jax.experimental.pallas import tpu_sc as plsc

D_MODEL = 2048
NUM_EXPERTS = 64
TOP_K = 2
BT = 1024
NS = 2
TOKENS = 16384
NW = 32
RPW = TOKENS // NW


def _probs_kernel(x1_ref, x2_ref, wt_ref, b_ref, probs_ref, probs_t_ref):
    wt = wt_ref[...]
    b = b_ref[...]
    for s, x_ref in enumerate((x1_ref, x2_ref)):
        logits = jax.lax.dot_general(
            x_ref[...], wt, (((1,), (1,)), ((), ())),
            preferred_element_type=jnp.float32,
            precision=jax.lax.Precision.DEFAULT,
        ) + b
        m = jnp.max(logits, axis=-1, keepdims=True)
        e = jnp.exp(logits - m)
        s_ = jnp.sum(e, axis=-1, keepdims=True)
        probs = e * (1.0 / s_)
        probs_ref[s * BT:(s + 1) * BT, :] = probs
        probs_t_ref[:, s * BT:(s + 1) * BT] = probs.T


def _tc_probs(x, W, b):
    tokens = x.shape[0]
    b2 = b.reshape(1, NUM_EXPERTS)
    return pl.pallas_call(
        _probs_kernel,
        grid=(tokens // (NS * BT),),
        in_specs=[
            pl.BlockSpec((BT, D_MODEL), lambda i: (NS * i, 0)),
            pl.BlockSpec((BT, D_MODEL), lambda i: (NS * i + 1, 0)),
            pl.BlockSpec((NUM_EXPERTS, D_MODEL), lambda i: (0, 0)),
            pl.BlockSpec((1, NUM_EXPERTS), lambda i: (0, 0)),
        ],
        out_specs=[
            pl.BlockSpec((NS * BT, NUM_EXPERTS), lambda i: (i, 0)),
            pl.BlockSpec((NUM_EXPERTS, NS * BT), lambda i: (0, i)),
        ],
        out_shape=[
            jax.ShapeDtypeStruct((tokens, NUM_EXPERTS), jnp.float32),
            jax.ShapeDtypeStruct((NUM_EXPERTS, tokens), jnp.float32),
        ],
    )(x, x, W, b2)


@functools.partial(
    pl.kernel,
    mesh=plsc.VectorSubcoreMesh(core_axis_name="c", subcore_axis_name="s"),
    out_type=[
        jax.ShapeDtypeStruct((TOP_K, TOKENS), jnp.float32),
        jax.ShapeDtypeStruct((TOP_K, TOKENS), jnp.int32),
    ],
    scratch_types=[
        pltpu.VMEM((NUM_EXPERTS, RPW), jnp.float32),
        pltpu.VMEM((TOP_K, RPW), jnp.float32),
        pltpu.VMEM((TOP_K, RPW), jnp.int32),
    ],
)
def _sc_top2(probs_t_hbm, tp_hbm, ti_hbm, probs_v, tp_v, ti_v):
    wid = lax.axis_index("s") * 2 + lax.axis_index("c")
    base = wid * RPW
    pltpu.sync_copy(probs_t_hbm.at[:, pl.ds(base, RPW)], probs_v)

    def chunk_body(j, carry):
        lo = j * 16
        best = probs_v[0, pl.ds(lo, 16)]
        ibest = jnp.zeros((16,), jnp.int32)
        second = jnp.full((16,), -1.0, jnp.float32)
        isecond = jnp.zeros((16,), jnp.int32)
        for e in range(1, NUM_EXPERTS):
            v = probs_v[e, pl.ds(lo, 16)]
            ev = jnp.full((16,), e, jnp.int32)
            gt1 = v > best
            gts = v > second
            isecond = jnp.where(gt1, ibest, jnp.where(gts, ev, isecond))
            second = jnp.where(gt1, best, jnp.where(gts, v, second))
            ibest = jnp.where(gt1, ev, ibest)
            best = jnp.where(gt1, v, best)
        denom = best + second + 1e-9
        tp_v[0, pl.ds(lo, 16)] = best / denom
        tp_v[1, pl.ds(lo, 16)] = second / denom
        ti_v[0, pl.ds(lo, 16)] = ibest
        ti_v[1, pl.ds(lo, 16)] = isecond
        return carry

    lax.fori_loop(0, RPW // 16, chunk_body, 0)
    pltpu.sync_copy(tp_v, tp_hbm.at[:, pl.ds(base, RPW)])
    pltpu.sync_copy(ti_v, ti_hbm.at[:, pl.ds(base, RPW)])


def kernel(x, W, b):
    probs, probs_t = _tc_probs(x, W, b)
    tp_t, ti_t = _sc_top2(probs_t)
    return (probs, tp_t.T, ti_t.T)
